# Initial kernel scaffold; baseline (speedup 1.0000x reference)
import jax
import jax.numpy as jnp
from jax.experimental import pallas as pl


def kernel(distance, similarities):
    raise NotImplementedError("write your pallas kernel here")


# speedup vs baseline: 1.1899x; 1.1899x over previous
import functools

import jax
import jax.numpy as jnp
from jax import lax
from jax.experimental import pallas as pl
from jax.experimental.pallas import tpu as pltpu
from jax.experimental.pallas import tpu_sc as plsc

_NUM_WORKERS = 32
_LANES = 16


def kernel(distance, similarities):
    B, I, J, K, D = distance.shape
    N = B * I * J
    PW = N // _NUM_WORKERS
    assert PW * _NUM_WORKERS == N and PW % _LANES == 0
    HALF = PW // 2

    dist = distance.reshape(N * K, D)
    sims = similarities.reshape(N * K)

    mesh = plsc.VectorSubcoreMesh(core_axis_name="c", subcore_axis_name="s")

    @functools.partial(
        pl.kernel,
        mesh=mesh,
        compiler_params=pltpu.CompilerParams(needs_layout_passes=False),
        out_type=[
            jax.ShapeDtypeStruct((N, D), jnp.float32),
            jax.ShapeDtypeStruct((N,), jnp.int32),
        ],
        scratch_types=[
            pltpu.VMEM((PW * K,), jnp.float32),
            pltpu.VMEM((PW,), jnp.int32),
            pltpu.VMEM((PW,), jnp.int32),
            pltpu.VMEM((PW, D), jnp.float32),
            pltpu.SemaphoreType.DMA,
        ],
    )
    def body(dist_hbm, sims_hbm, out_hbm, arg_hbm, sims_v, idx_v, arg_v, rows_v, sem):
        wid = lax.axis_index("s") * 2 + lax.axis_index("c")
        base = wid * PW
        pltpu.sync_copy(sims_hbm.at[pl.ds(base * K, PW * K)], sims_v)

        lane = lax.iota(jnp.int32, _LANES)
        for g in range(PW // _LANES):
            row0 = g * _LANES
            flat = (row0 + lane) * K
            best_v = plsc.load_gather(sims_v, [flat])
            best_i = jnp.zeros((_LANES,), jnp.int32)
            for k in range(1, K):
                v = plsc.load_gather(sims_v, [flat + k])
                m = v > best_v
                best_v = jnp.where(m, v, best_v)
                best_i = jnp.where(m, k, best_i)
            idx_v[pl.ds(row0, _LANES)] = (base + row0 + lane) * K + best_i
            arg_v[pl.ds(row0, _LANES)] = best_i

        copies = [
            pltpu.async_copy(
                dist_hbm.at[idx_v.at[pl.ds(j * HALF, HALF)]],
                rows_v.at[pl.ds(j * HALF, HALF)],
                sem,
            )
            for j in range(2)
        ]
        for c in copies:
            c.wait()

        pltpu.sync_copy(rows_v, out_hbm.at[pl.ds(base, PW)])
        pltpu.sync_copy(arg_v, arg_hbm.at[pl.ds(base, PW)])

    out, arg = body(dist, sims)
    return out.reshape(B, I, J, D), arg.reshape(B, I, J)
